# baseline (speedup 1.0000x reference)
import functools

import jax
import jax.numpy as jnp
from jax import lax
from jax.experimental import pallas as pl
from jax.experimental.pallas import tpu as pltpu
from jax.experimental.pallas import tpu_sc as plsc

_K = 20
_BLK = 256
_PW = 144
_CHUNK = 256


def _topk_body(h_ref, xb_ref, xa_ref, xt_ref, we1_ref,
               idx_ref, p_ref, *, n, d_h, d_x, hid, blk, n_total):
    b = pl.program_id(0)
    i = pl.program_id(1)

    h_blk = h_ref[...]
    x_blk = xb_ref[...]
    xt = xt_ref[...]

    d = jnp.zeros((blk, n), jnp.float32)
    for c in range(d_x):
        diff = x_blk[:, c:c + 1] - xt[c:c + 1, :]
        d = d + diff * diff
    row_ids = i * blk + lax.broadcasted_iota(jnp.int32, (blk, 1), 0)
    col_ids = lax.broadcasted_iota(jnp.int32, (blk, n), 1)
    d = jnp.where(col_ids == row_ids, d + 1e10, d)

    base = b * n
    for k in range(_K):
        d2 = jnp.min(d, axis=1, keepdims=True)
        is_min = d == d2
        idx = jnp.min(jnp.where(is_min, col_ids, n), axis=1, keepdims=True)
        idx_ref[:, k:k + 1] = idx + base
        d = jnp.where(col_ids == idx, 1e30, d)

    g_blk = jnp.dot(h_blk, we1_ref[pl.ds(d_h, d_h), :],
                    preferred_element_type=jnp.float32)
    pad = jnp.zeros((blk, _PW - hid - d_x), jnp.float32)
    p_ref[...] = jnp.concatenate([g_blk, x_blk, pad], axis=1)


def _sc_gather_body(table_hbm, idx_hbm, out_hbm, idx_v, rows_v, sem,
                    *, n_chunks, chunk, num_cores):
    wid = lax.axis_index("s") * num_cores + lax.axis_index("c")
    base = wid * (n_chunks * chunk)
    pltpu.sync_copy(idx_hbm.at[wid], idx_v)
    for j in range(n_chunks):
        pltpu.async_copy(table_hbm.at[idx_v.at[j]], rows_v, sem).wait()
        pltpu.sync_copy(rows_v, out_hbm.at[pl.ds(base + j * chunk, chunk)])


def _edge_body(g_ref, h_ref, x_ref, we1_ref, be1_ref, we2_ref, be2_ref,
               wh1_ref, bh1_ref, wh2_ref, bh2_ref, wct_ref, bc_ref,
               wst_ref, bs_ref, hout_ref, xout_ref, *, d_h, d_x, hid, blk):
    h_blk = h_ref[...]
    x_blk = x_ref[...]
    ge = g_ref[...]
    ne = blk * _K

    h1_blk = jnp.dot(h_blk, we1_ref[pl.ds(0, d_h), :],
                     preferred_element_type=jnp.float32)
    w_last = we1_ref[pl.ds(2 * d_h, 1), :]
    be1 = be1_ref[...]
    be2 = be2_ref[...]
    we2 = we2_ref[...]
    wct = wct_ref[...]
    bc = bc_ref[...]

    erow = lax.broadcasted_iota(jnp.int32, (ne, blk), 0) // _K
    ecol = lax.broadcasted_iota(jnp.int32, (ne, blk), 1)
    expand = (erow == ecol).astype(jnp.float32)

    g_j = ge[:, :hid]
    x_j = ge[:, hid:hid + d_x]
    x_i = jnp.dot(expand, x_blk, preferred_element_type=jnp.float32)
    h1_i = jnp.dot(expand, h1_blk, preferred_element_type=jnp.float32)
    x_diff = x_j - x_i
    d2 = jnp.sum(x_diff * x_diff, axis=1, keepdims=True)
    e = h1_i + g_j + d2 * w_last + be1
    s = e * jax.nn.sigmoid(e)
    m_ij = jnp.dot(s, we2, preferred_element_type=jnp.float32) + be2
    w = jnp.sum(m_ij * wct, axis=1, keepdims=True) + bc
    arow = lax.broadcasted_iota(jnp.int32, (blk, ne), 0)
    acol = lax.broadcasted_iota(jnp.int32, (blk, ne), 1) // _K
    agg = (arow == acol).astype(jnp.float32)
    m_acc = jnp.dot(agg, m_ij, preferred_element_type=jnp.float32)
    x_acc = jnp.dot(agg, w * x_diff, preferred_element_type=jnp.float32)

    t = (jnp.dot(h_blk, wh1_ref[pl.ds(0, d_h), :],
                 preferred_element_type=jnp.float32)
         + jnp.dot(m_acc, wh1_ref[pl.ds(d_h, hid), :],
                   preferred_element_type=jnp.float32)
         + bh1_ref[...])
    t = t * jax.nn.sigmoid(t)
    h_new = (jnp.dot(t, wh2_ref[...], preferred_element_type=jnp.float32)
             + bh2_ref[...] + h_blk)
    hout_ref[...] = h_new

    scale = jnp.tanh(jnp.sum(h_new * wst_ref[...], axis=1, keepdims=True)
                     + bs_ref[...])
    norm = jnp.sqrt(jnp.sum(x_acc * x_acc, axis=1, keepdims=True)) + 1e-8
    xout_ref[...] = x_blk + scale * (x_acc / norm) * 0.1


def kernel(h, x, W_e1, b_e1, W_e2, b_e2, W_h1, b_h1, W_h2, b_h2,
           W_c, b_c, W_s, b_s):
    b_sz, n, d_h = h.shape
    d_x = x.shape[-1]
    hid = W_e2.shape[0]
    blk = _BLK
    while n % blk:
        blk //= 2
    nb = n // blk
    nt = b_sz * n
    ne = nt * _K

    xt = jnp.swapaxes(x, 1, 2)

    topk = functools.partial(_topk_body, n=n, d_h=d_h, d_x=d_x, hid=hid,
                             blk=blk, n_total=nt)
    idx, ptab = pl.pallas_call(
        topk,
        grid=(b_sz, nb),
        in_specs=[
            pl.BlockSpec((None, blk, d_h), lambda b, i: (b, i, 0)),
            pl.BlockSpec((None, blk, d_x), lambda b, i: (b, i, 0)),
            pl.BlockSpec((None, n, d_x), lambda b, i: (b, 0, 0)),
            pl.BlockSpec((None, d_x, n), lambda b, i: (b, 0, 0)),
            pl.BlockSpec((2 * d_h + 1, hid), lambda b, i: (0, 0)),
        ],
        out_specs=[
            pl.BlockSpec((blk, _K), lambda b, i: (b * (n // blk) + i, 0)),
            pl.BlockSpec((blk, _PW), lambda b, i: (b * (n // blk) + i, 0)),
        ],
        out_shape=[
            jax.ShapeDtypeStruct((nt, _K), jnp.int32),
            jax.ShapeDtypeStruct((nt, _PW), jnp.float32),
        ],
    )(h, x, x, xt, W_e1)

    info = plsc.get_sparse_core_info()
    nw = info.num_cores * info.num_subcores
    rows_per_w = ne // nw
    n_chunks = rows_per_w // _CHUNK
    idx3 = idx.reshape(nw, n_chunks, _CHUNK)

    mesh = plsc.VectorSubcoreMesh(core_axis_name="c", subcore_axis_name="s")
    gather = functools.partial(_sc_gather_body, n_chunks=n_chunks,
                               chunk=_CHUNK, num_cores=info.num_cores)
    gfn = pl.kernel(
        gather,
        mesh=mesh,
        compiler_params=pltpu.CompilerParams(use_tc_tiling_on_sc=False),
        out_type=jax.ShapeDtypeStruct((ne, _PW), jnp.float32),
        scratch_types=[
            pltpu.VMEM((n_chunks, _CHUNK), jnp.int32),
            pltpu.VMEM((_CHUNK, _PW), jnp.float32),
            pltpu.SemaphoreType.DMA,
        ],
    )
    gout = gfn(ptab, idx3)

    h2 = h.reshape(nt, d_h)
    x2 = x.reshape(nt, d_x)
    be1 = b_e1.reshape(1, hid)
    be2 = b_e2.reshape(1, hid)
    bh1 = b_h1.reshape(1, d_h)
    bh2 = b_h2.reshape(1, d_h)
    wct = W_c.reshape(1, hid)
    bc = b_c.reshape(1, 1)
    wst = W_s.reshape(1, d_h)
    bs = b_s.reshape(1, 1)

    full = lambda shape: pl.BlockSpec(shape, lambda i: (0,) * len(shape))
    edge = functools.partial(_edge_body, d_h=d_h, d_x=d_x, hid=hid, blk=blk)
    h_new, x_new = pl.pallas_call(
        edge,
        grid=(nt // blk,),
        in_specs=[
            pl.BlockSpec((blk * _K, _PW), lambda i: (i, 0)),
            pl.BlockSpec((blk, d_h), lambda i: (i, 0)),
            pl.BlockSpec((blk, d_x), lambda i: (i, 0)),
            full((2 * d_h + 1, hid)), full((1, hid)),
            full((hid, hid)), full((1, hid)),
            full((d_h + hid, d_h)), full((1, d_h)),
            full((d_h, d_h)), full((1, d_h)),
            full((1, hid)), full((1, 1)),
            full((1, d_h)), full((1, 1)),
        ],
        out_specs=[
            pl.BlockSpec((blk, d_h), lambda i: (i, 0)),
            pl.BlockSpec((blk, d_x), lambda i: (i, 0)),
        ],
        out_shape=[
            jax.ShapeDtypeStruct((nt, d_h), jnp.float32),
            jax.ShapeDtypeStruct((nt, d_x), jnp.float32),
        ],
    )(gout, h2, x2, W_e1, be1, W_e2, be2, W_h1, bh1, W_h2, bh2,
      wct, bc, wst, bs)
    return h_new.reshape(b_sz, n, d_h), x_new.reshape(b_sz, n, d_x)


# speedup vs baseline: 1.1681x; 1.1681x over previous
import functools

import jax
import jax.numpy as jnp
from jax import lax
from jax.experimental import pallas as pl
from jax.experimental.pallas import tpu as pltpu
from jax.experimental.pallas import tpu_sc as plsc

_K = 20
_BLK = 256
_PW = 144
_CHUNK = 256


def _topk_body(h_ref, xb_ref, xa_ref, xt_ref, we1_ref,
               idx_ref, p_ref, *, n, d_h, d_x, hid, blk, n_total):
    b = pl.program_id(0)
    i = pl.program_id(1)

    h_blk = h_ref[...]
    x_blk = xb_ref[...]
    xt = xt_ref[...]

    d = jnp.zeros((blk, n), jnp.float32)
    for c in range(d_x):
        diff = x_blk[:, c:c + 1] - xt[c:c + 1, :]
        d = d + diff * diff
    row_ids = i * blk + lax.broadcasted_iota(jnp.int32, (blk, 1), 0)
    col_ids = lax.broadcasted_iota(jnp.int32, (blk, n), 1)
    d = jnp.where(col_ids == row_ids, d + 1e10, d)

    base = b * n
    for k in range(_K):
        d2 = jnp.min(d, axis=1, keepdims=True)
        is_min = d == d2
        idx = jnp.min(jnp.where(is_min, col_ids, n), axis=1, keepdims=True)
        idx_ref[:, k:k + 1] = idx + base
        d = jnp.where(col_ids == idx, 1e30, d)

    g_blk = jnp.dot(h_blk, we1_ref[pl.ds(d_h, d_h), :],
                    preferred_element_type=jnp.float32)
    pad = jnp.zeros((blk, _PW - hid - d_x), jnp.float32)
    p_ref[...] = jnp.concatenate([g_blk, x_blk, pad], axis=1)


def _sc_gather_body(table_hbm, idx_hbm, out_hbm, idx_v, rows_v, sem,
                    *, n_chunks, chunk, num_cores):
    wid = lax.axis_index("s") * num_cores + lax.axis_index("c")
    base = wid * (n_chunks * chunk)
    pltpu.sync_copy(idx_hbm.at[wid], idx_v)
    for j in range(n_chunks):
        pltpu.async_copy(table_hbm.at[idx_v.at[j]], rows_v, sem).wait()
        pltpu.sync_copy(rows_v, out_hbm.at[pl.ds(base + j * chunk, chunk)])


def _edge_body(g_ref, h_ref, x_ref, we1_ref, be1_ref, we2_ref, be2_ref,
               wh1_ref, bh1_ref, wh2_ref, bh2_ref, wct_ref, bc_ref,
               wst_ref, bs_ref, hout_ref, xout_ref, *, d_h, d_x, hid, blk):
    h_blk = h_ref[...]
    x_blk = x_ref[...]
    g3 = g_ref[...]

    h1_blk = jnp.dot(h_blk, we1_ref[pl.ds(0, d_h), :],
                     preferred_element_type=jnp.float32)
    w_last = we1_ref[pl.ds(2 * d_h, 1), :]
    be1 = be1_ref[...]
    be2 = be2_ref[...]
    we2 = we2_ref[...]
    wct = wct_ref[...]
    bc = bc_ref[...]

    m_acc = jnp.zeros((blk, hid), jnp.float32)
    x_acc = jnp.zeros((blk, d_x), jnp.float32)
    for k in range(_K):
        g_j = g3[k, :, :hid]
        x_j = g3[k, :, hid:hid + d_x]
        x_diff = x_j - x_blk
        d2 = jnp.sum(x_diff * x_diff, axis=1, keepdims=True)
        e = h1_blk + g_j + d2 * w_last + be1
        s = e * jax.nn.sigmoid(e)
        m_ij = jnp.dot(s, we2, preferred_element_type=jnp.float32) + be2
        m_acc = m_acc + m_ij
        w = jnp.sum(m_ij * wct, axis=1, keepdims=True) + bc
        x_acc = x_acc + w * x_diff

    t = (jnp.dot(h_blk, wh1_ref[pl.ds(0, d_h), :],
                 preferred_element_type=jnp.float32)
         + jnp.dot(m_acc, wh1_ref[pl.ds(d_h, hid), :],
                   preferred_element_type=jnp.float32)
         + bh1_ref[...])
    t = t * jax.nn.sigmoid(t)
    h_new = (jnp.dot(t, wh2_ref[...], preferred_element_type=jnp.float32)
             + bh2_ref[...] + h_blk)
    hout_ref[...] = h_new

    scale = jnp.tanh(jnp.sum(h_new * wst_ref[...], axis=1, keepdims=True)
                     + bs_ref[...])
    norm = jnp.sqrt(jnp.sum(x_acc * x_acc, axis=1, keepdims=True)) + 1e-8
    xout_ref[...] = x_blk + scale * (x_acc / norm) * 0.1


def kernel(h, x, W_e1, b_e1, W_e2, b_e2, W_h1, b_h1, W_h2, b_h2,
           W_c, b_c, W_s, b_s):
    b_sz, n, d_h = h.shape
    d_x = x.shape[-1]
    hid = W_e2.shape[0]
    blk = _BLK
    while n % blk:
        blk //= 2
    nb = n // blk
    nt = b_sz * n
    ne = nt * _K

    xt = jnp.swapaxes(x, 1, 2)

    topk = functools.partial(_topk_body, n=n, d_h=d_h, d_x=d_x, hid=hid,
                             blk=blk, n_total=nt)
    idx, ptab = pl.pallas_call(
        topk,
        grid=(b_sz, nb),
        in_specs=[
            pl.BlockSpec((None, blk, d_h), lambda b, i: (b, i, 0)),
            pl.BlockSpec((None, blk, d_x), lambda b, i: (b, i, 0)),
            pl.BlockSpec((None, n, d_x), lambda b, i: (b, 0, 0)),
            pl.BlockSpec((None, d_x, n), lambda b, i: (b, 0, 0)),
            pl.BlockSpec((2 * d_h + 1, hid), lambda b, i: (0, 0)),
        ],
        out_specs=[
            pl.BlockSpec((blk, _K), lambda b, i: (b * (n // blk) + i, 0)),
            pl.BlockSpec((blk, _PW), lambda b, i: (b * (n // blk) + i, 0)),
        ],
        out_shape=[
            jax.ShapeDtypeStruct((nt, _K), jnp.int32),
            jax.ShapeDtypeStruct((nt, _PW), jnp.float32),
        ],
    )(h, x, x, xt, W_e1)

    info = plsc.get_sparse_core_info()
    nw = info.num_cores * info.num_subcores
    rows_per_w = ne // nw
    n_chunks = rows_per_w // _CHUNK
    idx3 = jnp.transpose(idx).reshape(nw, n_chunks, _CHUNK)

    mesh = plsc.VectorSubcoreMesh(core_axis_name="c", subcore_axis_name="s")
    gather = functools.partial(_sc_gather_body, n_chunks=n_chunks,
                               chunk=_CHUNK, num_cores=info.num_cores)
    gfn = pl.kernel(
        gather,
        mesh=mesh,
        compiler_params=pltpu.CompilerParams(use_tc_tiling_on_sc=False),
        out_type=jax.ShapeDtypeStruct((ne, _PW), jnp.float32),
        scratch_types=[
            pltpu.VMEM((n_chunks, _CHUNK), jnp.int32),
            pltpu.VMEM((_CHUNK, _PW), jnp.float32),
            pltpu.SemaphoreType.DMA,
        ],
    )
    gout = gfn(ptab, idx3)
    g3 = gout.reshape(_K, nt, _PW)

    h2 = h.reshape(nt, d_h)
    x2 = x.reshape(nt, d_x)
    be1 = b_e1.reshape(1, hid)
    be2 = b_e2.reshape(1, hid)
    bh1 = b_h1.reshape(1, d_h)
    bh2 = b_h2.reshape(1, d_h)
    wct = W_c.reshape(1, hid)
    bc = b_c.reshape(1, 1)
    wst = W_s.reshape(1, d_h)
    bs = b_s.reshape(1, 1)

    full = lambda shape: pl.BlockSpec(shape, lambda i: (0,) * len(shape))
    edge = functools.partial(_edge_body, d_h=d_h, d_x=d_x, hid=hid, blk=blk)
    h_new, x_new = pl.pallas_call(
        edge,
        grid=(nt // blk,),
        in_specs=[
            pl.BlockSpec((_K, blk, _PW), lambda i: (0, i, 0)),
            pl.BlockSpec((blk, d_h), lambda i: (i, 0)),
            pl.BlockSpec((blk, d_x), lambda i: (i, 0)),
            full((2 * d_h + 1, hid)), full((1, hid)),
            full((hid, hid)), full((1, hid)),
            full((d_h + hid, d_h)), full((1, d_h)),
            full((d_h, d_h)), full((1, d_h)),
            full((1, hid)), full((1, 1)),
            full((1, d_h)), full((1, 1)),
        ],
        out_specs=[
            pl.BlockSpec((blk, d_h), lambda i: (i, 0)),
            pl.BlockSpec((blk, d_x), lambda i: (i, 0)),
        ],
        out_shape=[
            jax.ShapeDtypeStruct((nt, d_h), jnp.float32),
            jax.ShapeDtypeStruct((nt, d_x), jnp.float32),
        ],
    )(g3, h2, x2, W_e1, be1, W_e2, be2, W_h1, bh1, W_h2, bh2,
      wct, bc, wst, bs)
    return h_new.reshape(b_sz, n, d_h), x_new.reshape(b_sz, n, d_x)


# speedup vs baseline: 1.2242x; 1.0480x over previous
import functools

import jax
import jax.numpy as jnp
from jax import lax
from jax.experimental import pallas as pl
from jax.experimental.pallas import tpu as pltpu
from jax.experimental.pallas import tpu_sc as plsc

_K = 20
_BLK = 256
_PW = 144
_CHUNK = 256


def _topk_body(h_ref, xb_ref, xa_ref, xt_ref, we1_ref,
               idx_ref, p_ref, *, n, d_h, d_x, hid, blk, n_total):
    b = pl.program_id(0)
    i = pl.program_id(1)

    h_blk = h_ref[...]
    x_blk = xb_ref[...]
    xt = xt_ref[...]

    d = jnp.zeros((blk, n), jnp.float32)
    for c in range(d_x):
        diff = x_blk[:, c:c + 1] - xt[c:c + 1, :]
        d = d + diff * diff
    row_ids = i * blk + lax.broadcasted_iota(jnp.int32, (blk, 1), 0)
    col_ids = lax.broadcasted_iota(jnp.int32, (blk, n), 1)
    d = jnp.where(col_ids == row_ids, d + 1e10, d)

    base = b * n
    for k in range(_K):
        d2 = jnp.min(d, axis=1, keepdims=True)
        is_min = d == d2
        idx = jnp.min(jnp.where(is_min, col_ids, n), axis=1, keepdims=True)
        idx_ref[:, k:k + 1] = idx + base
        d = jnp.where(col_ids == idx, 1e30, d)

    g_blk = jnp.dot(h_blk, we1_ref[pl.ds(d_h, d_h), :],
                    preferred_element_type=jnp.float32)
    pad = jnp.zeros((blk, _PW - hid - d_x), jnp.float32)
    p_ref[...] = jnp.concatenate([g_blk, x_blk, pad], axis=1)


def _sc_gather_body(table_hbm, idx_hbm, out_hbm, idx_v, rows_v, sem,
                    *, n_chunks, chunk, num_cores):
    wid = lax.axis_index("s") * num_cores + lax.axis_index("c")
    base = wid * (n_chunks * chunk)
    pltpu.sync_copy(idx_hbm.at[wid], idx_v)
    for j in range(n_chunks):
        pltpu.async_copy(table_hbm.at[idx_v.at[j]], rows_v, sem).wait()
        pltpu.sync_copy(rows_v, out_hbm.at[pl.ds(base + j * chunk, chunk)])


def _edge_body(g_ref, h_ref, x_ref, we1_ref, be1_ref, we2_ref, be2_ref,
               wh1_ref, bh1_ref, wh2_ref, bh2_ref, wct_ref, bc_ref,
               wst_ref, bs_ref, hout_ref, xout_ref, *, d_h, d_x, hid, blk):
    h_blk = h_ref[...]
    x_blk = x_ref[...]
    g3 = g_ref[...]

    h1_blk = jnp.dot(h_blk, we1_ref[pl.ds(0, d_h), :],
                     preferred_element_type=jnp.float32)
    w_last = we1_ref[pl.ds(2 * d_h, 1), :]
    be1 = be1_ref[...]
    be2 = be2_ref[...]
    we2 = we2_ref[...]
    wct = wct_ref[...]
    bc = bc_ref[...]

    m_acc = jnp.zeros((blk, hid), jnp.float32)
    x_acc = jnp.zeros((blk, d_x), jnp.float32)
    for k in range(_K):
        g_j = g3[k, :, :hid]
        x_j = g3[k, :, hid:hid + d_x]
        x_diff = x_j - x_blk
        d2 = jnp.sum(x_diff * x_diff, axis=1, keepdims=True)
        e = h1_blk + g_j + d2 * w_last + be1
        s = e * jax.nn.sigmoid(e)
        m_ij = jnp.dot(s, we2, preferred_element_type=jnp.float32) + be2
        m_acc = m_acc + m_ij
        w = jnp.sum(m_ij * wct, axis=1, keepdims=True) + bc
        x_acc = x_acc + w * x_diff

    t = (jnp.dot(h_blk, wh1_ref[pl.ds(0, d_h), :],
                 preferred_element_type=jnp.float32)
         + jnp.dot(m_acc, wh1_ref[pl.ds(d_h, hid), :],
                   preferred_element_type=jnp.float32)
         + bh1_ref[...])
    t = t * jax.nn.sigmoid(t)
    h_new = (jnp.dot(t, wh2_ref[...], preferred_element_type=jnp.float32)
             + bh2_ref[...] + h_blk)
    hout_ref[...] = h_new

    scale = jnp.tanh(jnp.sum(h_new * wst_ref[...], axis=1, keepdims=True)
                     + bs_ref[...])
    norm = jnp.sqrt(jnp.sum(x_acc * x_acc, axis=1, keepdims=True)) + 1e-8
    xout_ref[...] = x_blk + scale * (x_acc / norm) * 0.1


def _pipeline(h, x, W_e1, b_e1, W_e2, b_e2, W_h1, b_h1, W_h2, b_h2,
              W_c, b_c, W_s, b_s):
    b_sz, n, d_h = h.shape
    d_x = x.shape[-1]
    hid = W_e2.shape[0]
    blk = _BLK
    while n % blk:
        blk //= 2
    nb = n // blk
    nt = b_sz * n
    ne = nt * _K

    xt = jnp.swapaxes(x, 1, 2)

    topk = functools.partial(_topk_body, n=n, d_h=d_h, d_x=d_x, hid=hid,
                             blk=blk, n_total=nt)
    idx, ptab = pl.pallas_call(
        topk,
        grid=(b_sz, nb),
        in_specs=[
            pl.BlockSpec((None, blk, d_h), lambda b, i: (b, i, 0)),
            pl.BlockSpec((None, blk, d_x), lambda b, i: (b, i, 0)),
            pl.BlockSpec((None, n, d_x), lambda b, i: (b, 0, 0)),
            pl.BlockSpec((None, d_x, n), lambda b, i: (b, 0, 0)),
            pl.BlockSpec((2 * d_h + 1, hid), lambda b, i: (0, 0)),
        ],
        out_specs=[
            pl.BlockSpec((blk, _K), lambda b, i: (b * (n // blk) + i, 0)),
            pl.BlockSpec((blk, _PW), lambda b, i: (b * (n // blk) + i, 0)),
        ],
        out_shape=[
            jax.ShapeDtypeStruct((nt, _K), jnp.int32),
            jax.ShapeDtypeStruct((nt, _PW), jnp.float32),
        ],
    )(h, x, x, xt, W_e1)

    info = plsc.get_sparse_core_info()
    nw = info.num_cores * info.num_subcores
    rows_per_w = ne // nw
    n_chunks = rows_per_w // _CHUNK
    idx3 = jnp.transpose(idx).reshape(nw, n_chunks, _CHUNK)

    mesh = plsc.VectorSubcoreMesh(core_axis_name="c", subcore_axis_name="s")
    gather = functools.partial(_sc_gather_body, n_chunks=n_chunks,
                               chunk=_CHUNK, num_cores=info.num_cores)
    gfn = pl.kernel(
        gather,
        mesh=mesh,
        compiler_params=pltpu.CompilerParams(use_tc_tiling_on_sc=False),
        out_type=jax.ShapeDtypeStruct((ne, _PW), jnp.float32),
        scratch_types=[
            pltpu.VMEM((n_chunks, _CHUNK), jnp.int32),
            pltpu.VMEM((_CHUNK, _PW), jnp.float32),
            pltpu.SemaphoreType.DMA,
        ],
    )
    gout = gfn(ptab, idx3)
    g3 = gout.reshape(_K, nt, _PW)

    h2 = h.reshape(nt, d_h)
    x2 = x.reshape(nt, d_x)
    be1 = b_e1.reshape(1, hid)
    be2 = b_e2.reshape(1, hid)
    bh1 = b_h1.reshape(1, d_h)
    bh2 = b_h2.reshape(1, d_h)
    wct = W_c.reshape(1, hid)
    bc = b_c.reshape(1, 1)
    wst = W_s.reshape(1, d_h)
    bs = b_s.reshape(1, 1)

    full = lambda shape: pl.BlockSpec(shape, lambda i: (0,) * len(shape))
    edge = functools.partial(_edge_body, d_h=d_h, d_x=d_x, hid=hid, blk=blk)
    h_new, x_new = pl.pallas_call(
        edge,
        grid=(nt // blk,),
        in_specs=[
            pl.BlockSpec((_K, blk, _PW), lambda i: (0, i, 0)),
            pl.BlockSpec((blk, d_h), lambda i: (i, 0)),
            pl.BlockSpec((blk, d_x), lambda i: (i, 0)),
            full((2 * d_h + 1, hid)), full((1, hid)),
            full((hid, hid)), full((1, hid)),
            full((d_h + hid, d_h)), full((1, d_h)),
            full((d_h, d_h)), full((1, d_h)),
            full((1, hid)), full((1, 1)),
            full((1, d_h)), full((1, 1)),
        ],
        out_specs=[
            pl.BlockSpec((blk, d_h), lambda i: (i, 0)),
            pl.BlockSpec((blk, d_x), lambda i: (i, 0)),
        ],
        out_shape=[
            jax.ShapeDtypeStruct((nt, d_h), jnp.float32),
            jax.ShapeDtypeStruct((nt, d_x), jnp.float32),
        ],
    )(g3, h2, x2, W_e1, be1, W_e2, be2, W_h1, bh1, W_h2, bh2,
      wct, bc, wst, bs)
    return h_new.reshape(b_sz, n, d_h), x_new.reshape(b_sz, n, d_x)


def kernel(h, x, *weights):
    b_sz = h.shape[0]
    if b_sz % 2:
        return _pipeline(h, x, *weights)
    hb = b_sz // 2
    h0, x0 = _pipeline(h[:hb], x[:hb], *weights)
    h1, x1 = _pipeline(h[hb:], x[hb:], *weights)
    return (jnp.concatenate([h0, h1], axis=0),
            jnp.concatenate([x0, x1], axis=0))
